# baseline (speedup 1.0000x reference)
import functools
import jax
import jax.numpy as jnp
import numpy as np
from jax import lax
from jax.experimental import pallas as pl
from jax.experimental.pallas import tpu as pltpu
from jax.experimental.pallas import tpu_sc as plsc

B = 4096
S = 100
NV = 104
NC, NS = 2, 16
NW = NC * NS
B_PER_W = B // NW


def _table_body(m_ref, out_ref):
    m = m_ref[...]
    v = lax.broadcasted_iota(jnp.int32, (NV, 1, 1), 0)
    out_ref[...] = (m < v).astype(jnp.float32)


def _build_table(m_const):
    return pl.pallas_call(
        _table_body,
        in_specs=[pl.BlockSpec((1, S, S), lambda: (0, 0, 0))],
        out_specs=pl.BlockSpec((NV, S, S), lambda: (0, 0, 0)),
        out_shape=jax.ShapeDtypeStruct((NV, S, S), jnp.float32),
    )(m_const)


def kernel(tensor_span):
    r = np.arange(S, dtype=np.int32)
    m_const = jnp.asarray(np.maximum.outer(r, r)[None])
    table = _build_table(m_const)
    spans = tensor_span.reshape(B)

    mesh = plsc.VectorSubcoreMesh(
        core_axis_name="c", subcore_axis_name="s", num_cores=NC, num_subcores=NS
    )

    @functools.partial(
        pl.kernel,
        mesh=mesh,
        compiler_params=pltpu.CompilerParams(use_tc_tiling_on_sc=True),
        out_type=jax.ShapeDtypeStruct((B, S, S), jnp.float32),
        scratch_types=[
            pltpu.VMEM((B_PER_W,), jnp.int32),
            pltpu.SemaphoreType.DMA,
        ],
    )
    def sc_copy(table_hbm, idx_hbm, out_hbm, idx_v, sem):
        sid = lax.axis_index("s")
        wid = sid * NC + lax.axis_index("c")
        base = wid * B_PER_W
        pltpu.sync_copy(idx_hbm.at[pl.ds(base, B_PER_W)], idx_v)

        def body(g, carry):
            sv = idx_v[pl.ds(g * 16, 16)]
            for half in range(2):
                handles = []
                for k in range(8):
                    kk = half * 8 + k
                    s = lax.min(lax.max(sv[kk], 0), NV - 1)
                    handles.append(
                        pltpu.async_copy(
                            table_hbm.at[s], out_hbm.at[base + g * 16 + kk], sem
                        )
                    )
                for h in handles:
                    h.wait()
            return carry

        lax.fori_loop(0, B_PER_W // 16, body, 0)

    return sc_copy(table, spans)


# speedup vs baseline: 16.6027x; 16.6027x over previous
import functools
import jax
import jax.numpy as jnp
import numpy as np
from jax import lax
from jax.experimental import pallas as pl
from jax.experimental.pallas import tpu as pltpu
from jax.experimental.pallas import tpu_sc as plsc

B = 4096
S = 100
NV = 104
NC, NS = 2, 16
NW = NC * NS
B_PER_W = B // NW
NG = B_PER_W // 16


def _table_body(m_ref, out_ref):
    m = m_ref[...]
    v = lax.broadcasted_iota(jnp.int32, (NV, 1, 1), 0)
    out_ref[...] = (m < v).astype(jnp.float32)


def _build_table(m_const):
    return pl.pallas_call(
        _table_body,
        in_specs=[pl.BlockSpec((1, S, S), lambda: (0, 0, 0))],
        out_specs=pl.BlockSpec((NV, S, S), lambda: (0, 0, 0)),
        out_shape=jax.ShapeDtypeStruct((NV, S, S), jnp.float32),
    )(m_const)


def kernel(tensor_span):
    r = np.arange(S, dtype=np.int32)
    m_const = jnp.asarray(np.maximum.outer(r, r)[None])
    table = _build_table(m_const)
    spans = tensor_span.reshape(B)

    mesh = plsc.VectorSubcoreMesh(
        core_axis_name="c", subcore_axis_name="s", num_cores=NC, num_subcores=NS
    )

    @functools.partial(
        pl.kernel,
        mesh=mesh,
        compiler_params=pltpu.CompilerParams(use_tc_tiling_on_sc=True),
        out_type=jax.ShapeDtypeStruct((B, S, S), jnp.float32),
        scratch_types=[
            pltpu.VMEM((B_PER_W,), jnp.int32),
            pltpu.VMEM((S, S), jnp.float32),
            pltpu.VMEM((S, S), jnp.float32),
            pltpu.SemaphoreType.DMA,
            pltpu.SemaphoreType.DMA,
        ],
    )
    def sc_copy(table_hbm, idx_hbm, out_hbm, idx_v, buf0, buf1, sem0, sem1):
        sid = lax.axis_index("s")
        wid = sid * NC + lax.axis_index("c")
        base = wid * B_PER_W
        pltpu.sync_copy(idx_hbm.at[pl.ds(base, B_PER_W)], idx_v)
        bufs = (buf0, buf1)
        sems = (sem0, sem1)

        def body(g, carry):
            sv = idx_v[pl.ds(g * 16, 16)]
            for k in range(16):
                p = k % 2
                t = g * 16 + k

                @pl.when(t >= 2)
                def _():
                    pltpu.make_async_copy(
                        bufs[p], out_hbm.at[base], sems[p]
                    ).wait()

                s = lax.min(lax.max(sv[k], 0), NV - 1)
                pltpu.sync_copy(table_hbm.at[s], bufs[p])
                pltpu.async_copy(bufs[p], out_hbm.at[base + t], sems[p])
            return carry

        lax.fori_loop(0, NG, body, 0)
        pltpu.make_async_copy(buf0, out_hbm.at[base], sem0).wait()
        pltpu.make_async_copy(buf1, out_hbm.at[base], sem1).wait()

    return sc_copy(table, spans)


# speedup vs baseline: 22.3902x; 1.3486x over previous
import functools
import jax
import jax.numpy as jnp
import numpy as np
from jax import lax
from jax.experimental import pallas as pl
from jax.experimental.pallas import tpu as pltpu
from jax.experimental.pallas import tpu_sc as plsc

B = 4096
S = 100
NV = 128
RPT = NV // 16
NC, NS = 2, 16
B_PER_C = B // NC
NG = B_PER_C // 16
CAP = 12


def _table_body(m_ref, out_ref):
    m = m_ref[...]
    v = lax.broadcasted_iota(jnp.int32, (NV, 1, 1), 0)
    out_ref[...] = (m < v).astype(jnp.float32)


def _build_table(m_const):
    return pl.pallas_call(
        _table_body,
        in_specs=[pl.BlockSpec((1, S, S), lambda: (0, 0, 0))],
        out_specs=pl.BlockSpec((NV, S, S), lambda: (0, 0, 0)),
        out_shape=jax.ShapeDtypeStruct((NV, S, S), jnp.float32),
    )(m_const)


def kernel(tensor_span):
    r = np.arange(S, dtype=np.int32)
    m_const = jnp.asarray(np.maximum.outer(r, r)[None])
    table = _build_table(m_const)
    spans = tensor_span.reshape(B)

    mesh = plsc.VectorSubcoreMesh(
        core_axis_name="c", subcore_axis_name="s", num_cores=NC, num_subcores=NS
    )

    @functools.partial(
        pl.kernel,
        mesh=mesh,
        compiler_params=pltpu.CompilerParams(use_tc_tiling_on_sc=True),
        out_type=jax.ShapeDtypeStruct((B, S, S), jnp.float32),
        scratch_types=[
            pltpu.VMEM((B_PER_C,), jnp.int32),
            pltpu.VMEM((RPT, S, S), jnp.float32),
            pltpu.SemaphoreType.DMA,
        ],
    )
    def sc_scatter(table_hbm, idx_hbm, out_hbm, idx_v, rows_v, sem):
        sid = lax.axis_index("s")
        cid = lax.axis_index("c")
        cbase = cid * B_PER_C
        lo = sid * RPT

        pltpu.sync_copy(table_hbm.at[pl.ds(lo, RPT)], rows_v)
        pltpu.sync_copy(idx_hbm.at[pl.ds(cbase, B_PER_C)], idx_v)

        def _drain(cnt):
            for j in range(16):
                @pl.when(j < cnt)
                def _():
                    pltpu.make_async_copy(
                        rows_v.at[0], out_hbm.at[cbase], sem
                    ).wait()

        def body(g, prev_cnt):
            sv = idx_v[pl.ds(g * 16, 16)]
            cnt = jnp.int32(0)
            for k in range(16):
                s = lax.min(lax.max(sv[k], 0), NV - 1)
                hit = (s >= lo) & (s < lo + RPT)

                @pl.when(hit)
                def _():
                    pltpu.async_copy(
                        rows_v.at[s - lo], out_hbm.at[cbase + g * 16 + k], sem
                    )

                cnt = cnt + jnp.where(hit, 1, 0)
            _drain(prev_cnt)
            return cnt

        last_cnt = lax.fori_loop(0, NG, body, jnp.int32(0))
        _drain(last_cnt)

    return sc_scatter(table, spans)


# speedup vs baseline: 22.6199x; 1.0103x over previous
import functools
import jax
import jax.numpy as jnp
import numpy as np
from jax import lax
from jax.experimental import pallas as pl
from jax.experimental.pallas import tpu as pltpu
from jax.experimental.pallas import tpu_sc as plsc

B = 4096
S = 100
NV = 128
RPT = NV // 16
NC, NS = 2, 16
B_PER_C = B // NC
NG = B_PER_C // 16
CAP = 12


def _table_body(m_ref, out_ref):
    m = m_ref[...]
    v = lax.broadcasted_iota(jnp.int32, (NV, 1, 1), 0)
    out_ref[...] = (m < v).astype(jnp.float32)


def _build_table(m_const):
    return pl.pallas_call(
        _table_body,
        in_specs=[pl.BlockSpec((1, S, S), lambda: (0, 0, 0))],
        out_specs=pl.BlockSpec((NV, S, S), lambda: (0, 0, 0)),
        out_shape=jax.ShapeDtypeStruct((NV, S, S), jnp.float32),
    )(m_const)


def kernel(tensor_span):
    r = np.arange(S, dtype=np.int32)
    m_const = jnp.asarray(np.maximum.outer(r, r)[None])
    table = _build_table(m_const)
    spans = tensor_span.reshape(B)

    mesh = plsc.VectorSubcoreMesh(
        core_axis_name="c", subcore_axis_name="s", num_cores=NC, num_subcores=NS
    )

    @functools.partial(
        pl.kernel,
        mesh=mesh,
        compiler_params=pltpu.CompilerParams(use_tc_tiling_on_sc=True),
        out_type=jax.ShapeDtypeStruct((B, S, S), jnp.float32),
        scratch_types=[
            pltpu.VMEM((B_PER_C,), jnp.int32),
            pltpu.VMEM((RPT, S, S), jnp.float32),
            pltpu.SemaphoreType.DMA,
        ],
    )
    def sc_scatter(table_hbm, idx_hbm, out_hbm, idx_v, rows_v, sem):
        sid = lax.axis_index("s")
        cid = lax.axis_index("c")
        cbase = cid * B_PER_C
        lo = sid * RPT

        pltpu.sync_copy(table_hbm.at[pl.ds(lo, RPT)], rows_v)
        pltpu.sync_copy(idx_hbm.at[pl.ds(cbase, B_PER_C)], idx_v)

        def _drain(cnt):
            for j in range(16):
                @pl.when(j < cnt)
                def _():
                    pltpu.make_async_copy(
                        rows_v.at[0], out_hbm.at[cbase], sem
                    ).wait()

        def body(g, carry):
            c0, c1, c2, c3 = carry
            sv = idx_v[pl.ds(g * 16, 16)]
            cnt = jnp.int32(0)
            for k in range(16):
                s = lax.min(lax.max(sv[k], 0), NV - 1)
                hit = (s >= lo) & (s < lo + RPT)

                @pl.when(hit)
                def _():
                    pltpu.async_copy(
                        rows_v.at[s - lo], out_hbm.at[cbase + g * 16 + k], sem
                    )

                cnt = cnt + jnp.where(hit, 1, 0)
            _drain(c0)
            return (c1, c2, c3, cnt)

        z = jnp.int32(0)
        tail = lax.fori_loop(0, NG, body, (z, z, z, z))
        for c in tail:
            _drain(c)

    return sc_scatter(table, spans)
